# baseline (speedup 1.0000x reference)
import jax
import jax.numpy as jnp
from jax.experimental import pallas as pl
from jax.experimental.pallas import tpu as pltpu

_HZ_TO_K = 6.62607015e-34 / 1.380649e-23


def _body(scale_ref, down_ref, up_ref, e_ref, o_ref):
    x = e_ref[...] * scale_ref[0, 0]
    m = jnp.max(x, axis=-1, keepdims=True)
    e = jnp.exp(x - m)
    denom = jnp.sum(e, axis=-1)
    iota = jax.lax.broadcasted_iota(jnp.int32, down_ref.shape + (e.shape[-1],), 2)
    d1h = (iota == down_ref[...][:, :, None]).astype(jnp.float32)
    u1h = (iota == up_ref[...][:, :, None]).astype(jnp.float32)
    num = jnp.sum(e * (d1h - u1h), axis=-1)
    o_ref[...] = num / denom


def kernel(energies, lvl_down, lvl_up, temperature):
    B, M, L = energies.shape
    scale = (-_HZ_TO_K / temperature.astype(jnp.float32)).reshape(1, 1)
    down = lvl_down.astype(jnp.int32).reshape(1, M)
    up = lvl_up.astype(jnp.int32).reshape(1, M)

    B_BLK = 64
    grid = (B // B_BLK,)
    return pl.pallas_call(
        _body,
        grid=grid,
        in_specs=[
            pl.BlockSpec(memory_space=pltpu.SMEM),
            pl.BlockSpec((1, M), lambda i: (0, 0)),
            pl.BlockSpec((1, M), lambda i: (0, 0)),
            pl.BlockSpec((B_BLK, M, L), lambda i: (i, 0, 0)),
        ],
        out_specs=pl.BlockSpec((B_BLK, M), lambda i: (i, 0)),
        out_shape=jax.ShapeDtypeStruct((B, M), jnp.float32),
    )(scale, down, up, energies)
